# Initial kernel scaffold; baseline (speedup 1.0000x reference)
import jax
import jax.numpy as jnp
from jax.experimental import pallas as pl


def kernel(x, beat_numbers):
    raise NotImplementedError("write your pallas kernel here")


# speedup vs baseline: 1.6712x; 1.6712x over previous
import functools

import jax
import jax.numpy as jnp
from jax import lax
from jax.experimental import pallas as pl
from jax.experimental.pallas import tpu as pltpu
from jax.experimental.pallas import tpu_sc as plsc

N_NOTES = 32768
D_FEAT = 64
N_BEATS = 1024
QPM_PRIMO_IDX = 4
TEMPO_IDX = 26

L = 16
NC = 2
NS = 16
NW = NC * NS

NOTES_PER_TILE = N_NOTES // NS
SCAN_STEPS = NOTES_PER_TILE // L
BEATS_PER_TILE = N_BEATS // NW
GROUP = 128
SENTINEL = N_NOTES - 1


def _body(x_hbm, bn_hbm, out_hbm,
          bnbuf, prevbuf, head, local, stage, fidx, rows, row0, outv, shared,
          sem):
    c = lax.axis_index("c")
    s = lax.axis_index("s")
    wid = c * NS + s
    base = s * NOTES_PER_TILE
    iota = lax.iota(jnp.int32, L)
    zeros = jnp.zeros((L,), jnp.int32)

    pltpu.sync_copy(bn_hbm.at[pl.ds(base, NOTES_PER_TILE)], bnbuf)

    @pl.when(s == 0)
    def _():
        prevbuf[...] = jnp.full((L,), -1, jnp.int32)

    @pl.when(s > 0)
    def _():
        pltpu.sync_copy(bn_hbm.at[pl.ds(base - L, L)], prevbuf)

    pltpu.sync_copy(bn_hbm.at[pl.ds(0, L)], head)
    bn0 = plsc.load_gather(head, [zeros])

    def init_step(i, carry):
        local[pl.ds(i * L, L)] = jnp.full((L,), SENTINEL, jnp.int32)
        return carry

    lax.fori_loop(0, N_BEATS // L, init_step, 0)

    cur = bnbuf[pl.ds(0, L)]
    prev = plsc.load_gather(bnbuf, [jnp.maximum(iota - 1, 0)])
    first_note = plsc.load_gather(bnbuf, [zeros])
    pred_note = plsc.load_gather(prevbuf, [jnp.full((L,), L - 1, jnp.int32)])
    lead_boundary = (first_note != pred_note) | (s == 0)
    boundary = (cur != prev) | ((iota == 0) & lead_boundary)
    plsc.store_scatter(local, [cur - bn0], base + iota, mask=boundary)

    def scan_step(k, carry):
        kcur = bnbuf[pl.ds(k * L, L)]
        kprev = plsc.load_gather(bnbuf, [k * L + iota - 1])
        plsc.store_scatter(local, [kcur - bn0], base + k * L + iota,
                           mask=kcur != kprev)
        return carry

    lax.fori_loop(1, SCAN_STEPS, scan_step, 0)

    pltpu.sync_copy(local, shared.at[pl.ds(s * N_BEATS, N_BEATS)])
    plsc.subcore_barrier()

    gb = (wid // (GROUP // BEATS_PER_TILE)) * GROUP
    off = (wid % (GROUP // BEATS_PER_TILE)) * BEATS_PER_TILE
    for t in range(NS):
        pltpu.sync_copy(shared.at[pl.ds(t * N_BEATS + gb, GROUP)],
                        stage.at[pl.ds(t * GROUP, GROUP)])
    for j in range(BEATS_PER_TILE // L):
        m = stage[pl.ds(off + j * L, L)]
        for t in range(1, NS):
            m = jnp.minimum(m, stage[pl.ds(t * GROUP + off + j * L, L)])
        fidx[pl.ds(j * L, L)] = m

    pltpu.async_copy(x_hbm.at[fidx], rows, sem).wait()

    pltpu.sync_copy(x_hbm.at[0], row0)
    qpm = plsc.load_gather(row0, [jnp.full((L,), QPM_PRIMO_IDX, jnp.int32)])
    tp0 = plsc.load_gather(row0, [jnp.full((L,), D_FEAT - 2, jnp.int32)])
    tp1 = plsc.load_gather(row0, [jnp.full((L,), D_FEAT - 1, jnp.int32)])

    ch = iota % 4
    pattern = jnp.where(ch == 0, qpm, jnp.where(ch == 1, tp0, tp1))
    for m_i in range(BEATS_PER_TILE * 4 // L):
        outv[pl.ds(m_i * L, L)] = pattern
    for j in range(BEATS_PER_TILE // L):
        tv = plsc.load_gather(
            rows, [j * L + iota, jnp.full((L,), TEMPO_IDX, jnp.int32)])
        plsc.store_scatter(outv, [iota * 4 + (j * L * 4 + 3)], tv)

    pltpu.sync_copy(outv, out_hbm.at[pl.ds(wid * BEATS_PER_TILE * 4,
                                           BEATS_PER_TILE * 4)])


@functools.partial(
    pl.kernel,
    mesh=plsc.VectorSubcoreMesh(core_axis_name="c", subcore_axis_name="s"),
    compiler_params=pltpu.CompilerParams(needs_layout_passes=False,
                                         use_tc_tiling_on_sc=False),
    out_type=jax.ShapeDtypeStruct((N_BEATS * 4,), jnp.float32),
    scratch_types=[
        pltpu.VMEM((NOTES_PER_TILE,), jnp.int32),
        pltpu.VMEM((L,), jnp.int32),
        pltpu.VMEM((L,), jnp.int32),
        pltpu.VMEM((N_BEATS,), jnp.int32),
        pltpu.VMEM((NS * GROUP,), jnp.int32),
        pltpu.VMEM((BEATS_PER_TILE,), jnp.int32),
        pltpu.VMEM((BEATS_PER_TILE, D_FEAT), jnp.float32),
        pltpu.VMEM((D_FEAT,), jnp.float32),
        pltpu.VMEM((BEATS_PER_TILE * 4,), jnp.float32),
        pltpu.VMEM_SHARED((NS * N_BEATS,), jnp.int32),
        pltpu.SemaphoreType.DMA,
    ],
)
def _tempo_vec_selector(x_hbm, bn_hbm, out_hbm, *scratch):
    _body(x_hbm, bn_hbm, out_hbm, *scratch)


def kernel(x, beat_numbers):
    x2 = x.reshape(N_NOTES, D_FEAT)
    bn = beat_numbers.astype(jnp.int32)
    out = _tempo_vec_selector(x2, bn)
    return out.reshape(1, N_BEATS, 4)
